# baseline (speedup 1.0000x reference)
import jax
import jax.numpy as jnp
from jax import lax
from jax.experimental import pallas as pl
from jax.experimental.pallas import tpu as pltpu
from jax.experimental.pallas import tpu_sc as plsc

VOCAB = 100000
D = 128
W = 200
B = 1024

NC, NS = 2, 16
NW = NC * NS
ROWS_PER_W = B // NW
H = 2
WH = W // H
NBUF = 3
NSTG = 1


class _Once:
    def __init__(self, d):
        self.d = d
        self.done = False

    def wait(self):
        if not self.done:
            self.d.wait()
            self.done = True


def _copy(src, dst, sem, add=False):
    return _Once(pltpu.async_copy(src, dst, sem, add=add))


def _body(tab_hbm, tok_hbm, pos_hbm, out_hbm, idx_v, pos_s, stg_s, *scratch):
    bufs = list(scratch[:NBUF])
    sem_idx = scratch[NBUF]
    sem_pos = list(scratch[NBUF + 1:NBUF + 1 + NBUF])
    sem_gat = list(scratch[NBUF + 1 + NBUF:NBUF + 1 + 2 * NBUF])
    sem_out = list(scratch[NBUF + 1 + 2 * NBUF:NBUF + 1 + 3 * NBUF])
    base = NBUF + 1 + 3 * NBUF
    sem_stg = list(scratch[base:base + NSTG])
    sem_hop = list(scratch[base + NSTG:base + 2 * NSTG])

    sid = lax.axis_index("s")
    wid = sid * NC + lax.axis_index("c")

    @pl.when(sid == 0)
    def _load_pos():
        pltpu.sync_copy(pos_hbm, pos_s)

    d_idx = pltpu.async_copy(tok_hbm.at[pl.ds(wid * ROWS_PER_W, ROWS_PER_W)],
                             idx_v, sem_idx)
    plsc.subcore_barrier()

    d_pos = [None] * NBUF
    d_out = [None] * NBUF
    d_gat = [None] * NBUF
    d_stg = [None] * NSTG
    d_hop = [None] * NSTG

    def flush_stage(q):
        if d_stg[q] is not None:
            dsc, prow = d_stg[q]
            dsc.wait()
            d_hop[q] = _copy(stg_s.at[sid].at[q],
                             out_hbm.at[pl.ds(prow * W, W)], sem_hop[q])
            d_stg[q] = None

    def start_pos(jw):
        s = jw % NBUF
        if d_out[s] is not None:
            d_out[s].wait()
            d_out[s] = None
        d_pos[s] = _copy(pos_s, bufs[s], sem_pos[s])

    def start_gather(jw):
        s = jw % NBUF
        d_pos[s].wait()
        d_gat[s] = [
            _copy(tab_hbm.at[idx_v.at[jw].at[h]],
                  bufs[s].at[pl.ds(h * WH, WH)],
                  sem_gat[s], add=True)
            for h in range(H)
        ]

    def finish(jw):
        s = jw % NBUF
        for qq in range(NSTG):
            flush_stage(qq)
        for d in d_gat[s]:
            d.wait()
        row = wid * ROWS_PER_W + jw
        if jw % 2 == 0:
            d_out[s] = _copy(bufs[s], out_hbm.at[pl.ds(row * W, W)],
                             sem_out[s])
        else:
            q = (jw // 2) % NSTG
            if d_hop[q] is not None:
                d_hop[q].wait()
                d_hop[q] = None
            d = _copy(bufs[s], stg_s.at[sid].at[q], sem_stg[q])
            d_stg[q] = (d, row)
            d_out[s] = d

    for jw in range(NBUF - 1):
        d_pos[jw] = _copy(pos_s, bufs[jw], sem_pos[jw])
    d_idx.wait()
    start_gather(0)

    for j in range(ROWS_PER_W):
        if j + 1 < ROWS_PER_W:
            start_gather(j + 1)
        finish(j)
        if j + NBUF - 1 < ROWS_PER_W:
            start_pos(j + NBUF - 1)

    for q in range(NSTG):
        flush_stage(q)
    for q in range(NSTG):
        if d_hop[q] is not None:
            d_hop[q].wait()
    for s in range(NBUF):
        if d_out[s] is not None:
            d_out[s].wait()


def kernel(tokens, token_embedding, position_embedding, position_indices):
    del position_indices
    tokens3 = tokens.reshape(B, H, WH).astype(jnp.int32)
    mesh = plsc.VectorSubcoreMesh(
        core_axis_name="c", subcore_axis_name="s",
        num_cores=NC, num_subcores=NS,
    )
    out = pl.kernel(
        _body,
        out_type=jax.ShapeDtypeStruct((B * W, D), jnp.float32),
        mesh=mesh,
        scratch_types=[
            pltpu.VMEM((ROWS_PER_W, H, WH), jnp.int32),
            pltpu.VMEM_SHARED((W, D), jnp.float32),
            pltpu.VMEM_SHARED((NS, NSTG, W, D), jnp.float32),
        ] + [pltpu.VMEM((W, D), jnp.float32)] * NBUF
          + [pltpu.SemaphoreType.DMA] * (1 + 3 * NBUF + 2 * NSTG),
    )(token_embedding, tokens3, position_embedding)
    return out.reshape(B, W, D)


# speedup vs baseline: 1.0748x; 1.0748x over previous
import jax
import jax.numpy as jnp
from jax import lax
from jax.experimental import pallas as pl
from jax.experimental.pallas import tpu as pltpu
from jax.experimental.pallas import tpu_sc as plsc

VOCAB = 100000
D = 128
W = 200
B = 1024

NC, NS = 2, 16
NW = NC * NS
ROWS_PER_W = B // NW
H = 2
WH = W // H
NBUF = 4


def _body(tab_hbm, tok_hbm, pos_hbm, out_hbm, idx_v, pos_s, *scratch):
    bufs = list(scratch[:NBUF])
    sem_idx = scratch[NBUF]
    sem_pos = list(scratch[NBUF + 1:NBUF + 1 + NBUF])
    sem_gat = list(scratch[NBUF + 1 + NBUF:NBUF + 1 + 2 * NBUF])
    sem_out = list(scratch[NBUF + 1 + 2 * NBUF:NBUF + 1 + 3 * NBUF])

    sid = lax.axis_index("s")
    wid = lax.axis_index("c") * NS + sid

    @pl.when(sid == 0)
    def _load_pos():
        pltpu.sync_copy(pos_hbm, pos_s)

    d_idx = pltpu.async_copy(tok_hbm.at[pl.ds(wid * ROWS_PER_W, ROWS_PER_W)],
                             idx_v, sem_idx)
    plsc.subcore_barrier()

    d_pos = [None] * NBUF
    d_out = [None] * NBUF
    d_gat = [None] * NBUF

    def start_pos(jw):
        s = jw % NBUF
        if d_out[s] is not None:
            d_out[s].wait()
            d_out[s] = None
        d_pos[s] = pltpu.async_copy(pos_s, bufs[s], sem_pos[s])

    def start_gather(jw):
        s = jw % NBUF
        d_pos[s].wait()
        d_gat[s] = [
            pltpu.async_copy(
                tab_hbm.at[idx_v.at[jw].at[h]],
                bufs[s].at[pl.ds(h * WH, WH)],
                sem_gat[s], add=True)
            for h in range(H)
        ]

    def finish(jw):
        s = jw % NBUF
        for d in d_gat[s]:
            d.wait()
        row = wid * ROWS_PER_W + jw
        d_out[s] = pltpu.async_copy(bufs[s], out_hbm.at[pl.ds(row * W, W)],
                                    sem_out[s])

    for jw in range(NBUF - 1):
        d_pos[jw] = pltpu.async_copy(pos_s, bufs[jw], sem_pos[jw])
    d_idx.wait()
    start_gather(0)

    for j in range(ROWS_PER_W):
        if j + 1 < ROWS_PER_W:
            start_gather(j + 1)
        if j + NBUF - 1 < ROWS_PER_W:
            start_pos(j + NBUF - 1)
        finish(j)

    for s in range(NBUF):
        if d_out[s] is not None:
            d_out[s].wait()


def kernel(tokens, token_embedding, position_embedding, position_indices):
    del position_indices
    tokens3 = tokens.reshape(B, H, WH).astype(jnp.int32)
    mesh = plsc.VectorSubcoreMesh(
        core_axis_name="c", subcore_axis_name="s",
        num_cores=NC, num_subcores=NS,
    )
    out = pl.kernel(
        _body,
        out_type=jax.ShapeDtypeStruct((B * W, D), jnp.float32),
        mesh=mesh,
        scratch_types=[
            pltpu.VMEM((ROWS_PER_W, H, WH), jnp.int32),
            pltpu.VMEM_SHARED((W, D), jnp.float32),
        ] + [pltpu.VMEM((W, D), jnp.float32)] * NBUF
          + [pltpu.SemaphoreType.DMA] * (1 + 3 * NBUF),
    )(token_embedding, tokens3, position_embedding)
    return out.reshape(B, W, D)
